# baseline (speedup 1.0000x reference)
import jax
import jax.numpy as jnp
from jax import lax
from jax.experimental import pallas as pl
from jax.experimental.pallas import tpu as pltpu
from jax.experimental.pallas import tpu_sc as plsc

NUM_CORES = 2
NUM_SUBCORES = 16
LANES = 16
NUM_WORKERS = NUM_CORES * NUM_SUBCORES
DIM = 32
BLK = 128
CH = 4
B_PER_W = 512
N_CHUNKS = B_PER_W // CH


def _sc_kernel(uidx_hbm, iidx_hbm, utab_hbm, itab_hbm, params_hbm, out_hbm,
               idx_u, idx_i, u_st0, u_st1, i_st0, i_st1,
               params_v, out_v, sem_u0, sem_u1, sem_i0, sem_i1):
    wid = lax.axis_index("s") * NUM_CORES + lax.axis_index("c")
    base = wid * B_PER_W

    pltpu.sync_copy(params_hbm, params_v)
    pltpu.sync_copy(uidx_hbm.at[pl.ds(base, B_PER_W)],
                    idx_u.at[pl.ds(0, B_PER_W)])
    pltpu.sync_copy(iidx_hbm.at[pl.ds(base, B_PER_W)],
                    idx_i.at[pl.ds(0, B_PER_W)])

    u_bufs = (u_st0, u_st1)
    i_bufs = (i_st0, i_st1)
    u_sems = (sem_u0, sem_u1)
    i_sems = (sem_i0, sem_i1)

    def issue_chunk(c, buf_id):
        ub = u_bufs[buf_id]
        ib = i_bufs[buf_id]
        iv_u = idx_u[pl.ds(c * CH, LANES)]
        iv_i = idx_i[pl.ds(c * CH, LANES)]
        for k in range(CH):
            bu = pl.multiple_of((iv_u[k] // BLK) * BLK, BLK)
            bi = pl.multiple_of((iv_i[k] // BLK) * BLK, BLK)
            pltpu.async_copy(utab_hbm.at[:, pl.ds(bu, BLK)],
                             ub.at[:, pl.ds(k * BLK, BLK)], u_sems[buf_id])
            pltpu.async_copy(itab_hbm.at[:, pl.ds(bi, BLK)],
                             ib.at[:, pl.ds(k * BLK, BLK)], i_sems[buf_id])

    def wait_chunk(buf_id):
        pltpu.make_async_copy(utab_hbm.at[:, pl.ds(0, CH * BLK)],
                              u_bufs[buf_id], u_sems[buf_id]).wait()
        pltpu.make_async_copy(itab_hbm.at[:, pl.ds(0, CH * BLK)],
                              i_bufs[buf_id], i_sems[buf_id]).wait()

    w_lo = params_v[pl.ds(0, LANES)]
    w_hi = params_v[pl.ds(LANES, LANES)]
    bias = params_v[pl.ds(DIM, LANES)][0]
    iota16 = lax.iota(jnp.int32, LANES)
    rows_lo = iota16
    rows_hi = iota16 + LANES

    def compute_chunk(c, buf_id, t, vals):
        ub = u_bufs[buf_id]
        ib = i_bufs[buf_id]
        iv_u = idx_u[pl.ds(c * CH, LANES)]
        iv_i = idx_i[pl.ds(c * CH, LANES)]
        for k in range(CH):
            cu = jnp.full((LANES,), (iv_u[k] % BLK) + k * BLK, jnp.int32)
            ci = jnp.full((LANES,), (iv_i[k] % BLK) + k * BLK, jnp.int32)
            u0 = plsc.load_gather(ub, [rows_lo, cu])
            u1 = plsc.load_gather(ub, [rows_hi, cu])
            i0 = plsc.load_gather(ib, [rows_lo, ci])
            i1 = plsc.load_gather(ib, [rows_hi, ci])
            s = jnp.sum(u0 * i0 * w_lo + u1 * i1 * w_hi)
            lane = (c * CH + k) % LANES
            vals = jnp.where(iota16 == lane, s, vals)
        return vals

    issue_chunk(0, 0)
    zeros = jnp.zeros((LANES,), jnp.float32)

    def pair_body(t, vals):
        c0 = 2 * t
        vals = jnp.where((t % 2) == 0, zeros, vals)
        issue_chunk(c0 + 1, 1)
        wait_chunk(0)
        vals = compute_chunk(c0, 0, t, vals)

        @pl.when(c0 + 2 < N_CHUNKS)
        def _():
            issue_chunk(c0 + 2, 0)

        wait_chunk(1)
        vals = compute_chunk(c0 + 1, 1, t, vals)

        @pl.when((t % 2) == 1)
        def _():
            out_v[pl.ds((t // 2) * LANES, LANES)] = vals + bias

        return vals

    lax.fori_loop(0, N_CHUNKS // 2, pair_body, zeros)

    pltpu.sync_copy(out_v, out_hbm.at[pl.ds(base, B_PER_W)])


def kernel(user_indices, item_indices, user_table, item_table, fc_w, fc_b):
    batch = user_indices.shape[0]
    params = jnp.concatenate(
        [fc_w.reshape(DIM), fc_b.reshape(1),
         jnp.zeros((15,), jnp.float32)]).astype(jnp.float32)

    mesh = plsc.VectorSubcoreMesh(core_axis_name="c", subcore_axis_name="s")
    run = pl.kernel(
        _sc_kernel,
        out_type=jax.ShapeDtypeStruct((batch,), jnp.float32),
        mesh=mesh,
        compiler_params=pltpu.CompilerParams(
            needs_layout_passes=False, use_tc_tiling_on_sc=True),
        scratch_types=[
            pltpu.VMEM((B_PER_W + LANES,), jnp.int32),
            pltpu.VMEM((B_PER_W + LANES,), jnp.int32),
            pltpu.VMEM((DIM, CH * BLK), jnp.float32),
            pltpu.VMEM((DIM, CH * BLK), jnp.float32),
            pltpu.VMEM((DIM, CH * BLK), jnp.float32),
            pltpu.VMEM((DIM, CH * BLK), jnp.float32),
            pltpu.VMEM((DIM + 16,), jnp.float32),
            pltpu.VMEM((B_PER_W,), jnp.float32),
            pltpu.SemaphoreType.DMA,
            pltpu.SemaphoreType.DMA,
            pltpu.SemaphoreType.DMA,
            pltpu.SemaphoreType.DMA,
        ],
    )
    return run(user_indices.astype(jnp.int32), item_indices.astype(jnp.int32),
               user_table.T, item_table.T, params)


# speedup vs baseline: 1.0910x; 1.0910x over previous
import jax
import jax.numpy as jnp
from jax import lax
from jax.experimental import pallas as pl
from jax.experimental.pallas import tpu as pltpu
from jax.experimental.pallas import tpu_sc as plsc

NUM_CORES = 2
NUM_SUBCORES = 16
LANES = 16
NUM_WORKERS = NUM_CORES * NUM_SUBCORES
DIM = 32
BLK = 128
CH = 2
NBUF = 4
B_PER_W = 512
N_CHUNKS = B_PER_W // CH


def _sc_kernel(uidx_hbm, iidx_hbm, utab_hbm, itab_hbm, params_hbm, out_hbm,
               idx_u, idx_i, u_st0, u_st1, u_st2, u_st3,
               i_st0, i_st1, i_st2, i_st3, params_v, out_v,
               sem_u0, sem_u1, sem_u2, sem_u3,
               sem_i0, sem_i1, sem_i2, sem_i3):
    wid = lax.axis_index("s") * NUM_CORES + lax.axis_index("c")
    base = wid * B_PER_W

    pltpu.sync_copy(params_hbm, params_v)
    pltpu.sync_copy(uidx_hbm.at[pl.ds(base, B_PER_W)],
                    idx_u.at[pl.ds(0, B_PER_W)])
    pltpu.sync_copy(iidx_hbm.at[pl.ds(base, B_PER_W)],
                    idx_i.at[pl.ds(0, B_PER_W)])

    u_bufs = (u_st0, u_st1, u_st2, u_st3)
    i_bufs = (i_st0, i_st1, i_st2, i_st3)
    u_sems = (sem_u0, sem_u1, sem_u2, sem_u3)
    i_sems = (sem_i0, sem_i1, sem_i2, sem_i3)

    def issue_chunk(c, buf_id):
        ub = u_bufs[buf_id]
        ib = i_bufs[buf_id]
        iv_u = idx_u[pl.ds(c * CH, LANES)]
        iv_i = idx_i[pl.ds(c * CH, LANES)]
        for k in range(CH):
            bu = pl.multiple_of((iv_u[k] // BLK) * BLK, BLK)
            bi = pl.multiple_of((iv_i[k] // BLK) * BLK, BLK)
            pltpu.async_copy(utab_hbm.at[:, pl.ds(bu, BLK)],
                             ub.at[:, pl.ds(k * BLK, BLK)], u_sems[buf_id])
            pltpu.async_copy(itab_hbm.at[:, pl.ds(bi, BLK)],
                             ib.at[:, pl.ds(k * BLK, BLK)], i_sems[buf_id])

    def wait_chunk(buf_id):
        pltpu.make_async_copy(utab_hbm.at[:, pl.ds(0, CH * BLK)],
                              u_bufs[buf_id], u_sems[buf_id]).wait()
        pltpu.make_async_copy(itab_hbm.at[:, pl.ds(0, CH * BLK)],
                              i_bufs[buf_id], i_sems[buf_id]).wait()

    w_lo = params_v[pl.ds(0, LANES)]
    w_hi = params_v[pl.ds(LANES, LANES)]
    bias = params_v[pl.ds(DIM, LANES)][0]
    iota16 = lax.iota(jnp.int32, LANES)
    rows_lo = iota16
    rows_hi = iota16 + LANES

    def compute_chunk(c, buf_id, j, vals):
        ub = u_bufs[buf_id]
        ib = i_bufs[buf_id]
        iv_u = idx_u[pl.ds(c * CH, LANES)]
        iv_i = idx_i[pl.ds(c * CH, LANES)]
        for k in range(CH):
            cu = jnp.full((LANES,), (iv_u[k] % BLK) + k * BLK, jnp.int32)
            ci = jnp.full((LANES,), (iv_i[k] % BLK) + k * BLK, jnp.int32)
            u0 = plsc.load_gather(ub, [rows_lo, cu])
            u1 = plsc.load_gather(ub, [rows_hi, cu])
            i0 = plsc.load_gather(ib, [rows_lo, ci])
            i1 = plsc.load_gather(ib, [rows_hi, ci])
            s = jnp.sum(u0 * i0 * w_lo + u1 * i1 * w_hi)
            vals = jnp.where(iota16 == (c * CH + k) % LANES, s, vals)
        return vals

    issue_chunk(0, 0)
    issue_chunk(1, 1)
    issue_chunk(2, 2)
    zeros = jnp.zeros((LANES,), jnp.float32)

    def body(t, vals):
        c0 = 4 * t
        vals = jnp.where((t % 2) == 0, zeros, vals)
        for j in range(4):
            c = c0 + j

            @pl.when(c + 3 < N_CHUNKS)
            def _():
                issue_chunk(c + 3, (j + 3) % 4)

            wait_chunk(j)
            vals = compute_chunk(c, j, j, vals)

        @pl.when((t % 2) == 1)
        def _():
            out_v[pl.ds((t // 2) * LANES, LANES)] = vals + bias

        return vals

    lax.fori_loop(0, N_CHUNKS // 4, body, zeros)

    pltpu.sync_copy(out_v, out_hbm.at[pl.ds(base, B_PER_W)])


def kernel(user_indices, item_indices, user_table, item_table, fc_w, fc_b):
    batch = user_indices.shape[0]
    params = jnp.concatenate(
        [fc_w.reshape(DIM), fc_b.reshape(1),
         jnp.zeros((15,), jnp.float32)]).astype(jnp.float32)

    mesh = plsc.VectorSubcoreMesh(core_axis_name="c", subcore_axis_name="s")
    stage = pltpu.VMEM((DIM, CH * BLK), jnp.float32)
    run = pl.kernel(
        _sc_kernel,
        out_type=jax.ShapeDtypeStruct((batch,), jnp.float32),
        mesh=mesh,
        compiler_params=pltpu.CompilerParams(
            needs_layout_passes=False, use_tc_tiling_on_sc=True),
        scratch_types=[
            pltpu.VMEM((B_PER_W + LANES,), jnp.int32),
            pltpu.VMEM((B_PER_W + LANES,), jnp.int32),
            stage, stage, stage, stage,
            stage, stage, stage, stage,
            pltpu.VMEM((DIM + 16,), jnp.float32),
            pltpu.VMEM((B_PER_W,), jnp.float32),
            pltpu.SemaphoreType.DMA, pltpu.SemaphoreType.DMA,
            pltpu.SemaphoreType.DMA, pltpu.SemaphoreType.DMA,
            pltpu.SemaphoreType.DMA, pltpu.SemaphoreType.DMA,
            pltpu.SemaphoreType.DMA, pltpu.SemaphoreType.DMA,
        ],
    )
    return run(user_indices.astype(jnp.int32), item_indices.astype(jnp.int32),
               user_table.T, item_table.T, params)
